# Initial kernel scaffold; baseline (speedup 1.0000x reference)
import jax
import jax.numpy as jnp
from jax.experimental import pallas as pl


def kernel(pred1, proj2, pred2, proj1):
    raise NotImplementedError("write your pallas kernel here")


# speedup vs baseline: 1.5846x; 1.5846x over previous
import functools

import jax
import jax.numpy as jnp
from jax import lax
from jax.experimental import pallas as pl
from jax.experimental.pallas import tpu as pltpu

_N = 50000
_D = 256
_BR = 1000
_GRID = _N // _BR
_EPS = 1e-12


def _pair_loss(p, q):
    sp = jnp.sum(p * p, axis=1)
    sq = jnp.sum(q * q, axis=1)
    d = jnp.sum(p * q, axis=1)
    mp = jnp.maximum(jnp.sqrt(sp), _EPS)
    mq = jnp.maximum(jnp.sqrt(sq), _EPS)
    rp = 1.0 / mp
    rq = 1.0 / mq
    return jnp.sum(sp * rp * rp + sq * rq * rq - 2.0 * d * rp * rq)


def _tc_body(p1, q2, p2, q1, out):
    i = pl.program_id(0)

    @pl.when(i == 0)
    def _():
        out[0, 0] = 0.0

    part = _pair_loss(p1[...], q2[...]) + _pair_loss(p2[...], q1[...])
    out[0, 0] += 0.5 * part


def kernel(pred1, proj2, pred2, proj1):
    in_spec = pl.BlockSpec((_BR, _D), lambda i: (i, 0))
    out = pl.pallas_call(
        _tc_body,
        grid=(_GRID,),
        in_specs=[in_spec] * 4,
        out_specs=pl.BlockSpec(memory_space=pltpu.SMEM),
        out_shape=jax.ShapeDtypeStruct((1, 1), jnp.float32),
    )(pred1, proj2, pred2, proj1)
    return out[0, 0]
